# baseline (speedup 1.0000x reference)
import functools

import jax
import jax.numpy as jnp
from jax import lax
from jax.experimental import pallas as pl
from jax.experimental.pallas import tpu as pltpu
from jax.experimental.pallas import tpu_sc as plsc

_N = 10000
_E = 320000
_DH = 128
_NCLS = 40
_D2P = 48
_DDEG = 8
_CHUNK = 128
_NCHUNK = _E // _CHUNK
_NW = 32
_STEPS = (_NCHUNK + _NW - 1) // _NW
_NTILE = 16
_RPT = 624
_TAIL0 = _RPT * _NTILE
_TAIL = _N - _TAIL0
_BR = 1000
_NJ = 80
_EPAD = _NW * _NJ * _CHUNK
_NDUMMY = 16
_NZ = _N + _NDUMMY
_K = 8
_SC_PARAMS = pltpu.CompilerParams(use_tc_tiling_on_sc=False)


def _pad_edges(src, dst):
    pad = _EPAD - _E
    fill = (jnp.arange(pad, dtype=jnp.int32) % _NDUMMY)
    src_p = jnp.concatenate([src, fill])
    dst_p = jnp.concatenate([dst, fill + _N])
    return src_p, dst_p


def _acc_zero_prologue(zeros_hbm, acc, sid):
    r0 = sid * _RPT
    pltpu.sync_copy(zeros_hbm.at[pl.ds(r0, _RPT)], acc.at[pl.ds(r0, _RPT)])

    @pl.when(sid == _NTILE - 1)
    def _():
        pltpu.sync_copy(zeros_hbm.at[pl.ds(_TAIL0, _NZ - _TAIL0)],
                        acc.at[pl.ds(_TAIL0, _NZ - _TAIL0)])


def _acc_flush_epilogue(acc, out_hbm, cid, sid):
    r0 = sid * _RPT
    pltpu.sync_copy(acc.at[pl.ds(r0, _RPT)], out_hbm.at[cid, pl.ds(r0, _RPT)])

    @pl.when(sid == _NTILE - 1)
    def _():
        pltpu.sync_copy(acc.at[pl.ds(_TAIL0, _TAIL)],
                        out_hbm.at[cid, pl.ds(_TAIL0, _TAIL)])


def _edge_loop(table, idx_s, idx_d, rows, gsems, ssems, acc, ng, k):
    def group(i, carry):
        gh = []
        for b in range(k):
            j = i * k + b

            @pl.when(i > 0)
            def _(b=b):
                pltpu.make_async_copy(
                    rows.at[b], acc.at[idx_d.at[0]], ssems.at[b]).wait()

            gh.append(pltpu.async_copy(
                table.at[idx_s.at[j]], rows.at[b], gsems.at[b]))
        for b in range(k):
            j = i * k + b
            gh[b].wait()
            pltpu.async_copy(rows.at[b], acc.at[idx_d.at[j]], ssems.at[b],
                             add=True)
        return carry

    lax.fori_loop(0, ng, group, 0)
    for b in range(k):
        pltpu.make_async_copy(rows.at[b], acc.at[idx_d.at[0]],
                              ssems.at[b]).wait()


_NJ_BIG = 88
_NJ_SMALL = 72


def _sc_scatter_add(vals, srcp, dstp, d, dtype=jnp.float32):
    mesh = plsc.VectorSubcoreMesh(core_axis_name="c", subcore_axis_name="s")
    zeros = jnp.zeros((_NZ, d), dtype)
    stage = d <= _D2P
    k = 8
    nj_buf = _NJ if stage else _NJ_BIG

    def body(vals_hbm, src_hbm, dst_hbm, zeros_hbm, out_hbm,
             idx_s, idx_d, rows, gsems, ssems, acc, *maybe_sp):
        cid = lax.axis_index("c")
        sid = lax.axis_index("s")
        _acc_zero_prologue(zeros_hbm, acc, sid)
        if stage:
            wid = sid * 2 + cid
            pltpu.sync_copy(src_hbm.at[wid], idx_s)
            pltpu.sync_copy(dst_hbm.at[wid], idx_d)
            vals_sp = maybe_sp[0]
            r0 = sid * _RPT
            pltpu.sync_copy(vals_hbm.at[pl.ds(r0, _RPT)],
                            vals_sp.at[pl.ds(r0, _RPT)])

            @pl.when(sid == _NTILE - 1)
            def _():
                pltpu.sync_copy(vals_hbm.at[pl.ds(_TAIL0, _TAIL)],
                                vals_sp.at[pl.ds(_TAIL0, _TAIL)])

            table = vals_sp
            ng = _NJ // k
        else:
            @pl.when(cid == 1)
            def _():
                s0 = sid * _NJ_BIG
                pltpu.sync_copy(src_hbm.at[pl.ds(s0, _NJ_BIG)], idx_s)
                pltpu.sync_copy(dst_hbm.at[pl.ds(s0, _NJ_BIG)], idx_d)

            @pl.when(cid == 0)
            def _():
                s0 = _NTILE * _NJ_BIG + sid * _NJ_SMALL
                pltpu.sync_copy(src_hbm.at[pl.ds(s0, _NJ_SMALL)],
                                idx_s.at[pl.ds(0, _NJ_SMALL)])
                pltpu.sync_copy(dst_hbm.at[pl.ds(s0, _NJ_SMALL)],
                                idx_d.at[pl.ds(0, _NJ_SMALL)])

            table = vals_hbm
            ng = jnp.where(cid == 1, _NJ_BIG // k, _NJ_SMALL // k)
        plsc.subcore_barrier()
        _edge_loop(table, idx_s, idx_d, rows, gsems, ssems, acc, ng, k)
        plsc.subcore_barrier()
        _acc_flush_epilogue(acc, out_hbm, cid, sid)

    scratch = [
        pltpu.VMEM((nj_buf, _CHUNK), jnp.int32),
        pltpu.VMEM((nj_buf, _CHUNK), jnp.int32),
        pltpu.VMEM((k, _CHUNK, d), dtype),
        pltpu.SemaphoreType.DMA((k,)),
        pltpu.SemaphoreType.DMA((k,)),
        pltpu.VMEM_SHARED((_NZ, d), dtype),
    ]
    if stage:
        scratch.append(pltpu.VMEM_SHARED((_N, d), dtype))
    f = pl.kernel(
        body,
        mesh=mesh,
        out_type=jax.ShapeDtypeStruct((2, _N, d), dtype),
        scratch_types=scratch,
        compiler_params=_SC_PARAMS,
    )
    return f(vals, srcp, dstp, zeros)


def _sc_degree(dstp):
    mesh = plsc.VectorSubcoreMesh(core_axis_name="c", subcore_axis_name="s")
    d = _DDEG
    zeros = jnp.zeros((_NZ, d), jnp.float32)
    ones = jnp.ones((_CHUNK, d), jnp.float32)

    def body(ones_hbm, dst_hbm, zeros_hbm, out_hbm, idx_d, ones_v, sem, acc):
        cid = lax.axis_index("c")
        sid = lax.axis_index("s")
        wid = sid * 2 + cid
        _acc_zero_prologue(zeros_hbm, acc, sid)
        pltpu.sync_copy(ones_hbm, ones_v)
        pltpu.sync_copy(dst_hbm.at[wid], idx_d)
        plsc.subcore_barrier()

        def step(j, carry):
            pltpu.async_copy(ones_v, acc.at[idx_d.at[j]], sem, add=True)
            return carry

        lax.fori_loop(0, _NJ, step, 0)

        def drain(j, carry):
            pltpu.make_async_copy(ones_v, acc.at[idx_d.at[0]], sem).wait()
            return carry

        lax.fori_loop(0, _NJ, drain, 0)
        plsc.subcore_barrier()
        _acc_flush_epilogue(acc, out_hbm, cid, sid)

    f = pl.kernel(
        body,
        mesh=mesh,
        out_type=jax.ShapeDtypeStruct((2, _N, d), jnp.float32),
        scratch_types=[
            pltpu.VMEM((_NJ, _CHUNK), jnp.int32),
            pltpu.VMEM((_CHUNK, d), jnp.float32),
            pltpu.SemaphoreType.DMA,
            pltpu.VMEM_SHARED((_NZ, d), jnp.float32),
        ],
        compiler_params=_SC_PARAMS,
    )
    return f(ones, dstp, zeros)


def _tc_layer1f(x, W1, degp):
    grid = (_N // _BR,)

    def body(x_ref, w_ref, dg_ref, hb_ref, dv_ref):
        dg = dg_ref[...]
        deg = dg[0, :, 0:1] + dg[1, :, 0:1] + 1.0
        dinv = lax.rsqrt(jnp.maximum(deg, 1.0))
        h = jnp.dot(x_ref[...], w_ref[...], preferred_element_type=jnp.float32)
        hb_ref[...] = (h * dinv).astype(jnp.bfloat16)
        dv_ref[...] = jnp.broadcast_to(dinv, (_BR, _DDEG))

    return pl.pallas_call(
        body,
        grid=grid,
        in_specs=[
            pl.BlockSpec((_BR, _DH), lambda i: (i, 0)),
            pl.BlockSpec((_DH, _DH), lambda i: (0, 0)),
            pl.BlockSpec((2, _BR, _DDEG), lambda i: (0, i, 0)),
        ],
        out_specs=[
            pl.BlockSpec((_BR, _DH), lambda i: (i, 0)),
            pl.BlockSpec((_BR, _DDEG), lambda i: (i, 0)),
        ],
        out_shape=[
            jax.ShapeDtypeStruct((_N, _DH), jnp.bfloat16),
            jax.ShapeDtypeStruct((_N, _DDEG), jnp.float32),
        ],
    )(x, W1, degp)


def _tc_matmul1(x, W1):
    grid = (_N // _BR,)

    def body(x_ref, w_ref, h_ref):
        h_ref[...] = jnp.dot(x_ref[...], w_ref[...],
                             preferred_element_type=jnp.float32)

    return pl.pallas_call(
        body,
        grid=grid,
        in_specs=[
            pl.BlockSpec((_BR, _DH), lambda i: (i, 0)),
            pl.BlockSpec((_DH, _DH), lambda i: (0, 0)),
        ],
        out_specs=pl.BlockSpec((_BR, _DH), lambda i: (i, 0)),
        out_shape=jax.ShapeDtypeStruct((_N, _DH), jnp.float32),
    )(x, W1)


def _tc_scale1(h1, degp):
    grid = (_N // _BR,)

    def body(h_ref, dg_ref, hb_ref, dv_ref):
        dg = dg_ref[...]
        deg = dg[0, :, 0:1] + dg[1, :, 0:1] + 1.0
        dinv = lax.rsqrt(jnp.maximum(deg, 1.0))
        hb_ref[...] = (h_ref[...] * dinv).astype(jnp.bfloat16)
        dv_ref[...] = jnp.broadcast_to(dinv, (_BR, _DDEG))

    return pl.pallas_call(
        body,
        grid=grid,
        in_specs=[
            pl.BlockSpec((_BR, _DH), lambda i: (i, 0)),
            pl.BlockSpec((2, _BR, _DDEG), lambda i: (0, i, 0)),
        ],
        out_specs=[
            pl.BlockSpec((_BR, _DH), lambda i: (i, 0)),
            pl.BlockSpec((_BR, _DDEG), lambda i: (i, 0)),
        ],
        out_shape=[
            jax.ShapeDtypeStruct((_N, _DH), jnp.bfloat16),
            jax.ShapeDtypeStruct((_N, _DDEG), jnp.float32),
        ],
    )(h1, degp)


def _tc_layer2(agg1, h1p, dinv8, b1, W2p):
    grid = (_N // _BR,)

    def body(agg_ref, h_ref, dv_ref, b_ref, w_ref, out_ref):
        dinv = dv_ref[...][:, 0:1]
        agg = agg_ref[...].astype(jnp.float32)
        h = h_ref[...].astype(jnp.float32)
        o1 = dinv * (agg[0] + agg[1] + h) + b_ref[...]
        x2 = jnp.maximum(o1, 0.0)
        u = jnp.dot(x2, w_ref[...], preferred_element_type=jnp.float32)
        out_ref[...] = (u * dinv).astype(jnp.bfloat16)

    return pl.pallas_call(
        body,
        grid=grid,
        in_specs=[
            pl.BlockSpec((2, _BR, _DH), lambda i: (0, i, 0)),
            pl.BlockSpec((_BR, _DH), lambda i: (i, 0)),
            pl.BlockSpec((_BR, _DDEG), lambda i: (i, 0)),
            pl.BlockSpec((1, _DH), lambda i: (0, 0)),
            pl.BlockSpec((_DH, _D2P), lambda i: (0, 0)),
        ],
        out_specs=pl.BlockSpec((_BR, _D2P), lambda i: (i, 0)),
        out_shape=jax.ShapeDtypeStruct((_N, _D2P), jnp.bfloat16),
    )(agg1, h1p, dinv8, b1, W2p)


def _tc_layer3(agg2, up, dinv8, b2p):
    grid = (_N // _BR,)

    def body(agg_ref, u_ref, dv_ref, b_ref, out_ref):
        dinv = dv_ref[...][:, 0:1]
        agg = agg_ref[...].astype(jnp.float32)
        u = u_ref[...].astype(jnp.float32)
        o2 = dinv * (agg[0] + agg[1] + u) + b_ref[...]
        col = lax.broadcasted_iota(jnp.int32, (_BR, _D2P), 1)
        real = col < _NCLS
        m = jnp.max(jnp.where(real, o2, -jnp.inf), axis=1, keepdims=True)
        e = jnp.where(real, jnp.exp(o2 - m), 0.0)
        s = jnp.sum(e, axis=1, keepdims=True)
        out_ref[...] = (o2 - m - jnp.log(s))[:, :_NCLS]

    return pl.pallas_call(
        body,
        grid=grid,
        in_specs=[
            pl.BlockSpec((2, _BR, _D2P), lambda i: (0, i, 0)),
            pl.BlockSpec((_BR, _D2P), lambda i: (i, 0)),
            pl.BlockSpec((_BR, _DDEG), lambda i: (i, 0)),
            pl.BlockSpec((1, _D2P), lambda i: (0, 0)),
        ],
        out_specs=pl.BlockSpec((_BR, _NCLS), lambda i: (i, 0)),
        out_shape=jax.ShapeDtypeStruct((_N, _NCLS), jnp.float32),
    )(agg2, up, dinv8, b2p)


def kernel(x, edge_index, W1, b1, W2, b2):
    ei = edge_index.astype(jnp.int32)
    src_f, dst_f = _pad_edges(ei[0], ei[1])
    srcp = src_f.reshape(_NW, _NJ, _CHUNK)
    dstp = dst_f.reshape(_NW, _NJ, _CHUNK)

    degp = _sc_degree(dstp)
    h1b, dinv8 = _tc_layer1f(x, W1, degp)
    srcc = src_f.reshape(_EPAD // _CHUNK, _CHUNK)
    dstc = dst_f.reshape(_EPAD // _CHUNK, _CHUNK)
    agg1 = _sc_scatter_add(h1b, srcc, dstc, _DH, jnp.bfloat16)

    W2p = jnp.pad(W2, ((0, 0), (0, _D2P - _NCLS)))
    up = _tc_layer2(agg1, h1b, dinv8, b1.reshape(1, _DH), W2p)
    agg2 = _sc_scatter_add(up, srcp, dstp, _D2P, jnp.bfloat16)

    b2p = jnp.concatenate(
        [b2, jnp.full((_D2P - _NCLS,), -1e30, jnp.float32)]).reshape(1, _D2P)
    return _tc_layer3(agg2, up, dinv8, b2p)


# speedup vs baseline: 1.0041x; 1.0041x over previous
import jax
import jax.numpy as jnp
from jax import lax
from jax.experimental import pallas as pl
from jax.experimental.pallas import tpu as pltpu
from jax.experimental.pallas import tpu_sc as plsc

_N = 10000
_E = 320000
_DH = 128
_NCLS = 40
_D2P = 48
_DDEG = 8
_CHUNK = 128
_NW = 32
_NTILE = 16
_RPT = 624
_TAIL0 = _RPT * _NTILE
_TAIL = _N - _TAIL0
_BR = 1000
_NJ = 80
_EPAD = _NW * _NJ * _CHUNK
_NDUMMY = 16
_NZ = _N + _NDUMMY
_SC_PARAMS = pltpu.CompilerParams(use_tc_tiling_on_sc=False)


def _pad_edges(src, dst):
    pad = _EPAD - _E
    fill = (jnp.arange(pad, dtype=jnp.int32) % _NDUMMY)
    src_p = jnp.concatenate([src, fill])
    dst_p = jnp.concatenate([dst, fill + _N])
    return src_p, dst_p


def _acc_zero_prologue(zeros_hbm, acc, sid):
    r0 = sid * _RPT
    pltpu.sync_copy(zeros_hbm.at[pl.ds(r0, _RPT)], acc.at[pl.ds(r0, _RPT)])

    @pl.when(sid == _NTILE - 1)
    def _():
        pltpu.sync_copy(zeros_hbm.at[pl.ds(_TAIL0, _NZ - _TAIL0)],
                        acc.at[pl.ds(_TAIL0, _NZ - _TAIL0)])


def _acc_flush_epilogue(acc, out_hbm, cid, sid):
    r0 = sid * _RPT
    pltpu.sync_copy(acc.at[pl.ds(r0, _RPT)], out_hbm.at[cid, pl.ds(r0, _RPT)])

    @pl.when(sid == _NTILE - 1)
    def _():
        pltpu.sync_copy(acc.at[pl.ds(_TAIL0, _TAIL)],
                        out_hbm.at[cid, pl.ds(_TAIL0, _TAIL)])


def _edge_loop(table, idx_s, idx_d, rows, gsems, ssems, acc, ng, k):
    def group(i, carry):
        gh = []
        for b in range(k):
            j = i * k + b

            @pl.when(i > 0)
            def _(b=b):
                pltpu.make_async_copy(
                    rows.at[b], acc.at[idx_d.at[0]], ssems.at[b]).wait()

            gh.append(pltpu.async_copy(
                table.at[idx_s.at[j]], rows.at[b], gsems.at[b]))
        for b in range(k):
            j = i * k + b
            gh[b].wait()
            pltpu.async_copy(rows.at[b], acc.at[idx_d.at[j]], ssems.at[b],
                             add=True)
        return carry

    lax.fori_loop(0, ng, group, 0)
    for b in range(k):
        pltpu.make_async_copy(rows.at[b], acc.at[idx_d.at[0]],
                              ssems.at[b]).wait()


_NJ_BIG = 88
_NJ_SMALL = 72


def _sc_scatter_add(vals, srcp, dstp, d, dtype=jnp.float32):
    mesh = plsc.VectorSubcoreMesh(core_axis_name="c", subcore_axis_name="s")
    zeros = jnp.zeros((_NZ, d), dtype)
    stage = d <= _D2P
    k = 8
    nj_buf = _NJ if stage else _NJ_BIG

    def body(vals_hbm, src_hbm, dst_hbm, zeros_hbm, out_hbm,
             idx_s, idx_d, rows, gsems, ssems, acc, *maybe_sp):
        cid = lax.axis_index("c")
        sid = lax.axis_index("s")
        _acc_zero_prologue(zeros_hbm, acc, sid)
        if stage:
            wid = sid * 2 + cid
            pltpu.sync_copy(src_hbm.at[wid], idx_s)
            pltpu.sync_copy(dst_hbm.at[wid], idx_d)
            vals_sp = maybe_sp[0]
            r0 = sid * _RPT
            pltpu.sync_copy(vals_hbm.at[pl.ds(r0, _RPT)],
                            vals_sp.at[pl.ds(r0, _RPT)])

            @pl.when(sid == _NTILE - 1)
            def _():
                pltpu.sync_copy(vals_hbm.at[pl.ds(_TAIL0, _TAIL)],
                                vals_sp.at[pl.ds(_TAIL0, _TAIL)])

            table = vals_sp
            ng = _NJ // k
        else:
            @pl.when(cid == 0)
            def _():
                s0 = sid * _NJ_BIG
                pltpu.sync_copy(src_hbm.at[pl.ds(s0, _NJ_BIG)], idx_s)
                pltpu.sync_copy(dst_hbm.at[pl.ds(s0, _NJ_BIG)], idx_d)

            @pl.when(cid == 1)
            def _():
                s0 = _NTILE * _NJ_BIG + sid * _NJ_SMALL
                pltpu.sync_copy(src_hbm.at[pl.ds(s0, _NJ_SMALL)],
                                idx_s.at[pl.ds(0, _NJ_SMALL)])
                pltpu.sync_copy(dst_hbm.at[pl.ds(s0, _NJ_SMALL)],
                                idx_d.at[pl.ds(0, _NJ_SMALL)])

            table = vals_hbm
            ng = jnp.where(cid == 0, _NJ_BIG // k, _NJ_SMALL // k)
        plsc.subcore_barrier()
        _edge_loop(table, idx_s, idx_d, rows, gsems, ssems, acc, ng, k)
        plsc.subcore_barrier()
        _acc_flush_epilogue(acc, out_hbm, cid, sid)

    scratch = [
        pltpu.VMEM((nj_buf, _CHUNK), jnp.int32),
        pltpu.VMEM((nj_buf, _CHUNK), jnp.int32),
        pltpu.VMEM((k, _CHUNK, d), dtype),
        pltpu.SemaphoreType.DMA((k,)),
        pltpu.SemaphoreType.DMA((k,)),
        pltpu.VMEM_SHARED((_NZ, d), dtype),
    ]
    if stage:
        scratch.append(pltpu.VMEM_SHARED((_N, d), dtype))
    f = pl.kernel(
        body,
        mesh=mesh,
        out_type=jax.ShapeDtypeStruct((2, _N, d), dtype),
        scratch_types=scratch,
        compiler_params=_SC_PARAMS,
    )
    return f(vals, srcp, dstp, zeros)


def _sc_degree(dstp):
    mesh = plsc.VectorSubcoreMesh(core_axis_name="c", subcore_axis_name="s")
    d = _DDEG
    zeros = jnp.zeros((_NZ, d), jnp.float32)
    ones = jnp.ones((_CHUNK, d), jnp.float32)

    def body(ones_hbm, dst_hbm, zeros_hbm, out_hbm, idx_d, ones_v, sem, acc):
        cid = lax.axis_index("c")
        sid = lax.axis_index("s")
        wid = sid * 2 + cid
        _acc_zero_prologue(zeros_hbm, acc, sid)
        pltpu.sync_copy(ones_hbm, ones_v)
        pltpu.sync_copy(dst_hbm.at[wid], idx_d)
        plsc.subcore_barrier()

        def step(j, carry):
            pltpu.async_copy(ones_v, acc.at[idx_d.at[j]], sem, add=True)
            return carry

        lax.fori_loop(0, _NJ, step, 0)

        def drain(j, carry):
            pltpu.make_async_copy(ones_v, acc.at[idx_d.at[0]], sem).wait()
            return carry

        lax.fori_loop(0, _NJ, drain, 0)
        plsc.subcore_barrier()
        _acc_flush_epilogue(acc, out_hbm, cid, sid)

    f = pl.kernel(
        body,
        mesh=mesh,
        out_type=jax.ShapeDtypeStruct((2, _N, d), jnp.float32),
        scratch_types=[
            pltpu.VMEM((_NJ, _CHUNK), jnp.int32),
            pltpu.VMEM((_CHUNK, d), jnp.float32),
            pltpu.SemaphoreType.DMA,
            pltpu.VMEM_SHARED((_NZ, d), jnp.float32),
        ],
        compiler_params=_SC_PARAMS,
    )
    return f(ones, dstp, zeros)


def _tc_layer1f(x, W1, degp):
    grid = (_N // _BR,)

    def body(x_ref, w_ref, dg_ref, hb_ref, dv_ref):
        dg = dg_ref[...]
        deg = dg[0, :, 0:1] + dg[1, :, 0:1] + 1.0
        dinv = lax.rsqrt(jnp.maximum(deg, 1.0))
        h = jnp.dot(x_ref[...], w_ref[...], preferred_element_type=jnp.float32)
        hb_ref[...] = (h * dinv).astype(jnp.bfloat16)
        dv_ref[...] = jnp.broadcast_to(dinv, (_BR, _DDEG))

    return pl.pallas_call(
        body,
        grid=grid,
        in_specs=[
            pl.BlockSpec((_BR, _DH), lambda i: (i, 0)),
            pl.BlockSpec((_DH, _DH), lambda i: (0, 0)),
            pl.BlockSpec((2, _BR, _DDEG), lambda i: (0, i, 0)),
        ],
        out_specs=[
            pl.BlockSpec((_BR, _DH), lambda i: (i, 0)),
            pl.BlockSpec((_BR, _DDEG), lambda i: (i, 0)),
        ],
        out_shape=[
            jax.ShapeDtypeStruct((_N, _DH), jnp.bfloat16),
            jax.ShapeDtypeStruct((_N, _DDEG), jnp.float32),
        ],
    )(x, W1, degp)


def _tc_layer2(agg1, h1p, dinv8, b1, W2p):
    grid = (_N // _BR,)

    def body(agg_ref, h_ref, dv_ref, b_ref, w_ref, out_ref):
        dinv = dv_ref[...][:, 0:1]
        agg = agg_ref[...].astype(jnp.float32)
        h = h_ref[...].astype(jnp.float32)
        o1 = dinv * (agg[0] + agg[1] + h) + b_ref[...]
        x2 = jnp.maximum(o1, 0.0)
        u = jnp.dot(x2, w_ref[...], preferred_element_type=jnp.float32)
        out_ref[...] = (u * dinv).astype(jnp.bfloat16)

    return pl.pallas_call(
        body,
        grid=grid,
        in_specs=[
            pl.BlockSpec((2, _BR, _DH), lambda i: (0, i, 0)),
            pl.BlockSpec((_BR, _DH), lambda i: (i, 0)),
            pl.BlockSpec((_BR, _DDEG), lambda i: (i, 0)),
            pl.BlockSpec((1, _DH), lambda i: (0, 0)),
            pl.BlockSpec((_DH, _D2P), lambda i: (0, 0)),
        ],
        out_specs=pl.BlockSpec((_BR, _D2P), lambda i: (i, 0)),
        out_shape=jax.ShapeDtypeStruct((_N, _D2P), jnp.bfloat16),
    )(agg1, h1p, dinv8, b1, W2p)


def _tc_layer3(agg2, up, dinv8, b2p):
    grid = (_N // _BR,)

    def body(agg_ref, u_ref, dv_ref, b_ref, out_ref):
        dinv = dv_ref[...][:, 0:1]
        agg = agg_ref[...].astype(jnp.float32)
        u = u_ref[...].astype(jnp.float32)
        o2 = dinv * (agg[0] + agg[1] + u) + b_ref[...]
        col = lax.broadcasted_iota(jnp.int32, (_BR, _D2P), 1)
        real = col < _NCLS
        m = jnp.max(jnp.where(real, o2, -jnp.inf), axis=1, keepdims=True)
        e = jnp.where(real, jnp.exp(o2 - m), 0.0)
        s = jnp.sum(e, axis=1, keepdims=True)
        out_ref[...] = (o2 - m - jnp.log(s))[:, :_NCLS]

    return pl.pallas_call(
        body,
        grid=grid,
        in_specs=[
            pl.BlockSpec((2, _BR, _D2P), lambda i: (0, i, 0)),
            pl.BlockSpec((_BR, _D2P), lambda i: (i, 0)),
            pl.BlockSpec((_BR, _DDEG), lambda i: (i, 0)),
            pl.BlockSpec((1, _D2P), lambda i: (0, 0)),
        ],
        out_specs=pl.BlockSpec((_BR, _NCLS), lambda i: (i, 0)),
        out_shape=jax.ShapeDtypeStruct((_N, _NCLS), jnp.float32),
    )(agg2, up, dinv8, b2p)


def kernel(x, edge_index, W1, b1, W2, b2):
    ei = edge_index.astype(jnp.int32)
    src_f, dst_f = _pad_edges(ei[0], ei[1])
    srcp = src_f.reshape(_NW, _NJ, _CHUNK)
    dstp = dst_f.reshape(_NW, _NJ, _CHUNK)

    degp = _sc_degree(dstp)
    h1b, dinv8 = _tc_layer1f(x, W1, degp)
    srcc = src_f.reshape(_EPAD // _CHUNK, _CHUNK)
    dstc = dst_f.reshape(_EPAD // _CHUNK, _CHUNK)
    agg1 = _sc_scatter_add(h1b, srcc, dstc, _DH, jnp.bfloat16)

    W2p = jnp.pad(W2, ((0, 0), (0, _D2P - _NCLS)))
    up = _tc_layer2(agg1, h1b, dinv8, b1.reshape(1, _DH), W2p)
    agg2 = _sc_scatter_add(up, srcp, dstp, _D2P, jnp.bfloat16)

    b2p = jnp.concatenate(
        [b2, jnp.full((_D2P - _NCLS,), -1e30, jnp.float32)]).reshape(1, _D2P)
    return _tc_layer3(agg2, up, dinv8, b2p)
